# baseline (speedup 1.0000x reference)
import functools

import jax
import jax.numpy as jnp
from jax.experimental import pallas as pl
from jax.experimental.pallas import tpu as pltpu

N = 10000
DEG = 32
D = 128
DOUT = 128
BN = 400
KSPLIT = 2
KB = DEG // KSPLIT


def _body(self_ref, neigh_ref, w_ref, out_ref, acc_ref):
    j = pl.program_id(1)
    part = jnp.sum(neigh_ref[...], axis=1)

    @pl.when(j == 0)
    def _init():
        acc_ref[...] = part

    @pl.when(j != 0)
    def _accum():
        acc_ref[...] += part

    @pl.when(j == KSPLIT - 1)
    def _finish():
        m = (acc_ref[...] + self_ref[...]) * (1.0 / (DEG + 1))
        out_ref[...] = jnp.maximum(
            jnp.dot(m, w_ref[...], preferred_element_type=jnp.float32), 0.0
        )


@jax.jit
def kernel(self_vecs, neigh_vecs, W):
    grid = (N // BN, KSPLIT)
    return pl.pallas_call(
        _body,
        grid=grid,
        in_specs=[
            pl.BlockSpec((BN, D), lambda i, j: (i, 0)),
            pl.BlockSpec((BN, KB, D), lambda i, j: (i, j, 0)),
            pl.BlockSpec((D, DOUT), lambda i, j: (0, 0)),
        ],
        out_specs=pl.BlockSpec((BN, DOUT), lambda i, j: (i, 0)),
        out_shape=jax.ShapeDtypeStruct((N, DOUT), jnp.float32),
        scratch_shapes=[pltpu.VMEM((BN, D), jnp.float32)],
        compiler_params=pltpu.CompilerParams(
            dimension_semantics=("parallel", "arbitrary"),
        ),
    )(self_vecs, neigh_vecs, W)


# speedup vs baseline: 1.3278x; 1.3278x over previous
import functools

import jax
import jax.numpy as jnp
from jax.experimental import pallas as pl
from jax.experimental.pallas import tpu as pltpu

N = 10000
DEG = 32
D = 128
DOUT = 128
BN = 400


def _body(self_ref, neigh_ref, w_ref, out_ref):
    s = jnp.sum(neigh_ref[...], axis=1) + self_ref[...]
    m = s * (1.0 / (DEG + 1))
    out_ref[...] = jnp.maximum(
        jnp.dot(m, w_ref[...], preferred_element_type=jnp.float32), 0.0
    )


@jax.jit
def kernel(self_vecs, neigh_vecs, W):
    grid = (N // BN,)
    return pl.pallas_call(
        _body,
        grid=grid,
        in_specs=[
            pl.BlockSpec((BN, D), lambda i: (i, 0)),
            pl.BlockSpec((BN, DEG, D), lambda i: (i, 0, 0)),
            pl.BlockSpec((D, DOUT), lambda i: (0, 0)),
        ],
        out_specs=pl.BlockSpec((BN, DOUT), lambda i: (i, 0)),
        out_shape=jax.ShapeDtypeStruct((N, DOUT), jnp.float32),
        compiler_params=pltpu.CompilerParams(
            dimension_semantics=("parallel",),
        ),
    )(self_vecs, neigh_vecs, W)
